# baseline (speedup 1.0000x reference)
import functools

import jax
import jax.numpy as jnp
from jax import lax
from jax.experimental import pallas as pl
from jax.experimental.pallas import tpu as pltpu
from jax.experimental.pallas import tpu_sc as plsc

_NC = 2
_NS = 16
_NW = _NC * _NS

_IDXW = 128
_NBUF = 4
_DELAY = 2
_LANES = 16


def _scale_table_body(w_ref, o_ref):
    o_ref[...] = (
        jnp.maximum(w_ref[...], 0.0) * w_ref.shape[-1] ** 0.5
    ).astype(jnp.bfloat16)


def _scaled_table_bf16(w):
    v, d = w.shape
    bs = 1024
    grid = (v + bs - 1) // bs
    return pl.pallas_call(
        _scale_table_body,
        grid=(grid,),
        in_specs=[pl.BlockSpec((bs, d), lambda i: (i, 0))],
        out_specs=pl.BlockSpec((bs, d), lambda i: (i, 0)),
        out_shape=jax.ShapeDtypeStruct((v, d), jnp.bfloat16),
    )(w)


@functools.partial(jax.jit, static_argnums=(2, 3))
def _sc_gather(table_bf16, idx2, n_rows, d):
    rows_per_w = n_rows // _NW
    n_chunks = rows_per_w // _IDXW
    irows_per_w = n_chunks
    _PRO = _NBUF + _DELAY + (n_chunks - _NBUF - _DELAY) % _NBUF
    assert n_chunks > _PRO and (n_chunks - _PRO) % _NBUF == 0

    mesh = plsc.VectorSubcoreMesh(core_axis_name="c", subcore_axis_name="s")

    sems = [pltpu.SemaphoreType.DMA] * (2 * _NBUF)

    @functools.partial(
        pl.kernel,
        mesh=mesh,
        compiler_params=pltpu.CompilerParams(use_tc_tiling_on_sc=False, needs_layout_passes=False),
        out_type=jax.ShapeDtypeStruct((n_rows, d), jnp.float32),
        scratch_types=[
            pltpu.VMEM((irows_per_w, _IDXW), jnp.int32),
            pltpu.VMEM((_NBUF, _IDXW, d), jnp.bfloat16),
            pltpu.VMEM((_NBUF, _IDXW, d), jnp.float32),
        ] + sems,
    )
    def k(table_hbm, idx_hbm, out_hbm, idx_all, bf_v, f32_v, *all_sems):
        g_sem = all_sems[:_NBUF]
        o_sem = all_sems[_NBUF:]
        wid = lax.axis_index("s") * _NC + lax.axis_index("c")
        irow0 = wid * irows_per_w

        pltpu.sync_copy(idx_hbm.at[pl.ds(irow0, irows_per_w)], idx_all)

        def fire_gather(b, c):
            pltpu.async_copy(
                table_hbm.at[idx_all.at[c]], bf_v.at[b], g_sem[b]
            )

        def wait_gather(b):
            pltpu.make_async_copy(
                table_hbm.at[idx_all.at[0]], bf_v.at[b], g_sem[b]
            ).wait()

        def fire_out(b, c):
            pltpu.async_copy(
                f32_v.at[b],
                out_hbm.at[pl.ds((irow0 + c) * _IDXW, _IDXW)],
                o_sem[b],
            )

        def wait_out(b):
            pltpu.make_async_copy(
                f32_v.at[b], out_hbm.at[pl.ds(0, _IDXW)], o_sem[b]
            ).wait()

        ngrp = d // (2 * _LANES)

        def unpack_rows(b):
            @plsc.parallel_loop(0, _IDXW, unroll=4)
            def row_fn(r):
                for g in range(ngrp):
                    w = bf_v[b, r, pl.ds(g * 2 * _LANES, 2 * _LANES)]
                    ev, od = plsc.unpack(
                        w,
                        format=plsc.PackFormat.INTERLEAVED,
                        preferred_element_type=jnp.float32,
                    )
                    f32_v[b, r, pl.ds(g * 2 * _LANES, _LANES)] = ev
                    f32_v[b, r, pl.ds(g * 2 * _LANES + _LANES, _LANES)] = od

        def step(c, b_static, fire_g, do_cw, do_wait_o):
            if fire_g:
                fire_gather(b_static, c)
            if do_cw:
                b2 = (b_static - _DELAY) % _NBUF
                if do_wait_o:
                    wait_out(b2)
                wait_gather(b2)
                unpack_rows(b2)
                fire_out(b2, c - _DELAY)

        for c in range(_PRO):
            step(c, c % _NBUF, True, c >= _DELAY, c >= _NBUF + _DELAY)

        def body(i, carry):
            for kk in range(_NBUF):
                c = _PRO + i * _NBUF + kk
                step(c, (_PRO + kk) % _NBUF, True, True, True)
            return carry

        lax.fori_loop(0, (n_chunks - _PRO) // _NBUF, body, 0)

        for c in range(n_chunks, n_chunks + _DELAY):
            step(c, c % _NBUF, False, True, True)
        for b in range(_NBUF):
            wait_out(b)

    return k(table_bf16, idx2)


def kernel(x, embed_weight):
    b, h = x.shape
    v, d = embed_weight.shape
    n_rows = b * h
    table_bf16 = _scaled_table_bf16(embed_weight)
    table_perm = (
        table_bf16.reshape(v, d // 32, 2, 16)
        .swapaxes(2, 3)
        .reshape(v, d)
    )
    idx2 = x.reshape(n_rows // _IDXW, _IDXW).astype(jnp.int32)
    out = _sc_gather(table_perm, idx2, n_rows, d)
    return out.reshape(b, h, d)


# speedup vs baseline: 1.7720x; 1.7720x over previous
import functools

import jax
import jax.numpy as jnp
from jax import lax
from jax.experimental import pallas as pl
from jax.experimental.pallas import tpu as pltpu
from jax.experimental.pallas import tpu_sc as plsc

_NC = 2
_NS = 16
_NW = _NC * _NS

_IDXW = 128
_NBUF = 5
_DELAY = 2


def _scale_table_body(w_ref, o_ref):
    o_ref[...] = jnp.maximum(w_ref[...], 0.0) * o_ref.shape[-1] ** 0.5


def _scaled_table(w):
    v, d = w.shape
    bs = 1024
    grid = (v + bs - 1) // bs
    return pl.pallas_call(
        _scale_table_body,
        grid=(grid,),
        in_specs=[pl.BlockSpec((bs, d), lambda i: (i, 0))],
        out_specs=pl.BlockSpec((bs, d), lambda i: (i, 0)),
        out_shape=jax.ShapeDtypeStruct((v, d), w.dtype),
    )(w)


@functools.partial(jax.jit, static_argnums=(2, 3))
def _sc_gather(table, idx2, n_rows, d):
    rows_per_w = n_rows // _NW
    n_chunks = rows_per_w // _IDXW
    irows_per_w = n_chunks
    assert n_chunks % _NBUF == 0 and n_chunks >= 2 * _NBUF

    mesh = plsc.VectorSubcoreMesh(core_axis_name="c", subcore_axis_name="s")

    sems = [pltpu.SemaphoreType.DMA] * (2 * _NBUF)

    @functools.partial(
        pl.kernel,
        mesh=mesh,
        out_type=jax.ShapeDtypeStruct((n_rows, d), jnp.float32),
        scratch_types=[
            pltpu.VMEM((irows_per_w, _IDXW), jnp.int32),
            pltpu.VMEM((_NBUF, _IDXW, d), jnp.float32),
        ] + sems,
    )
    def k(table_hbm, idx_hbm, out_hbm, idx_all, rows_v, *all_sems):
        g_sem = all_sems[:_NBUF]
        o_sem = all_sems[_NBUF:]
        wid = lax.axis_index("s") * _NC + lax.axis_index("c")
        irow0 = wid * irows_per_w

        pltpu.sync_copy(idx_hbm.at[pl.ds(irow0, irows_per_w)], idx_all)

        def fire_gather(b, c):
            pltpu.async_copy(
                table_hbm.at[idx_all.at[c]], rows_v.at[b], g_sem[b]
            )

        def wait_gather(b):
            pltpu.make_async_copy(
                table_hbm.at[idx_all.at[0]], rows_v.at[b], g_sem[b]
            ).wait()

        def fire_out(b, c):
            pltpu.async_copy(
                rows_v.at[b],
                out_hbm.at[pl.ds((irow0 + c) * _IDXW, _IDXW)],
                o_sem[b],
            )

        def wait_out(b):
            pltpu.make_async_copy(
                rows_v.at[b], out_hbm.at[pl.ds(0, _IDXW)], o_sem[b]
            ).wait()

        scale = float(d) ** 0.5

        def relu_scale(b):
            def row_fn(r, carry):
                for j in range(d // 16):
                    v = rows_v[b, r, pl.ds(j * 16, 16)]
                    rows_v[b, r, pl.ds(j * 16, 16)] = (
                        jnp.maximum(v, 0.0) * scale
                    )
                return carry

            lax.fori_loop(0, _IDXW, row_fn, 0)

        def step(c, k_static, fire_g, wait_g, wait_o):
            if wait_o:
                wait_out(k_static)
            if fire_g:
                fire_gather(k_static, c)
            if wait_g:
                b2 = (k_static - _DELAY) % _NBUF
                wait_gather(b2)
                relu_scale(b2)
                fire_out(b2, c - _DELAY)

        for c in range(_NBUF):
            step(c, c, True, c >= _DELAY, False)

        def body(i, carry):
            for kk in range(_NBUF):
                step(_NBUF + i * _NBUF + kk, kk, True, True, True)
            return carry

        lax.fori_loop(0, n_chunks // _NBUF - 1, body, 0)

        for c in range(n_chunks, n_chunks + _DELAY):
            step(c, c % _NBUF, False, True, False)
        for b in range(_NBUF):
            wait_out(b)

    return k(table, idx2)


def kernel(x, embed_weight):
    b, h = x.shape
    v, d = embed_weight.shape
    n_rows = b * h
    idx2 = x.reshape(n_rows // _IDXW, _IDXW).astype(jnp.int32)
    out = _sc_gather(embed_weight, idx2, n_rows, d)
    return out.reshape(b, h, d)


# speedup vs baseline: 1.7742x; 1.0013x over previous
import functools

import jax
import jax.numpy as jnp
from jax import lax
from jax.experimental import pallas as pl
from jax.experimental.pallas import tpu as pltpu
from jax.experimental.pallas import tpu_sc as plsc

_NC = 2
_NS = 16
_NW = _NC * _NS

_IDXW = 128
_NBUF = 5
_DELAY = 3


def _scale_table_body(w_ref, o_ref):
    o_ref[...] = jnp.maximum(w_ref[...], 0.0) * o_ref.shape[-1] ** 0.5


def _scaled_table(w):
    v, d = w.shape
    bs = 1024
    grid = (v + bs - 1) // bs
    return pl.pallas_call(
        _scale_table_body,
        grid=(grid,),
        in_specs=[pl.BlockSpec((bs, d), lambda i: (i, 0))],
        out_specs=pl.BlockSpec((bs, d), lambda i: (i, 0)),
        out_shape=jax.ShapeDtypeStruct((v, d), w.dtype),
    )(w)


@functools.partial(jax.jit, static_argnums=(2, 3))
def _sc_gather(table, idx2, n_rows, d):
    rows_per_w = n_rows // _NW
    n_chunks = rows_per_w // _IDXW
    irows_per_w = n_chunks
    assert n_chunks % _NBUF == 0 and n_chunks >= 2 * _NBUF

    mesh = plsc.VectorSubcoreMesh(core_axis_name="c", subcore_axis_name="s")

    sems = [pltpu.SemaphoreType.DMA] * (2 * _NBUF)

    @functools.partial(
        pl.kernel,
        mesh=mesh,
        out_type=jax.ShapeDtypeStruct((n_rows, d), jnp.float32),
        scratch_types=[
            pltpu.VMEM((irows_per_w, _IDXW), jnp.int32),
            pltpu.VMEM((_NBUF, _IDXW, d), jnp.float32),
        ] + sems,
    )
    def k(table_hbm, idx_hbm, out_hbm, idx_all, rows_v, *all_sems):
        g_sem = all_sems[:_NBUF]
        o_sem = all_sems[_NBUF:]
        wid = lax.axis_index("s") * _NC + lax.axis_index("c")
        irow0 = wid * irows_per_w

        pltpu.sync_copy(idx_hbm.at[pl.ds(irow0, irows_per_w)], idx_all)

        def fire_gather(b, c):
            pltpu.async_copy(
                table_hbm.at[idx_all.at[c]], rows_v.at[b], g_sem[b]
            )

        def wait_gather(b):
            pltpu.make_async_copy(
                table_hbm.at[idx_all.at[0]], rows_v.at[b], g_sem[b]
            ).wait()

        def fire_out(b, c):
            pltpu.async_copy(
                rows_v.at[b],
                out_hbm.at[pl.ds((irow0 + c) * _IDXW, _IDXW)],
                o_sem[b],
            )

        def wait_out(b):
            pltpu.make_async_copy(
                rows_v.at[b], out_hbm.at[pl.ds(0, _IDXW)], o_sem[b]
            ).wait()

        scale = float(d) ** 0.5

        def relu_scale(b):
            def row_fn(r, carry):
                for j in range(d // 16):
                    v = rows_v[b, r, pl.ds(j * 16, 16)]
                    rows_v[b, r, pl.ds(j * 16, 16)] = (
                        jnp.maximum(v, 0.0) * scale
                    )
                return carry

            lax.fori_loop(0, _IDXW, row_fn, 0)

        def step(c, k_static, fire_g, wait_g, wait_o):
            if wait_o:
                wait_out(k_static)
            if fire_g:
                fire_gather(k_static, c)
            if wait_g:
                b2 = (k_static - _DELAY) % _NBUF
                wait_gather(b2)
                relu_scale(b2)
                fire_out(b2, c - _DELAY)

        for c in range(_NBUF):
            step(c, c, True, c >= _DELAY, False)

        def body(i, carry):
            for kk in range(_NBUF):
                step(_NBUF + i * _NBUF + kk, kk, True, True, True)
            return carry

        lax.fori_loop(0, n_chunks // _NBUF - 1, body, 0)

        for c in range(n_chunks, n_chunks + _DELAY):
            step(c, c % _NBUF, False, True, False)
        for b in range(_NBUF):
            wait_out(b)

    return k(table, idx2)


def kernel(x, embed_weight):
    b, h = x.shape
    v, d = embed_weight.shape
    n_rows = b * h
    idx2 = x.reshape(n_rows // _IDXW, _IDXW).astype(jnp.int32)
    out = _sc_gather(embed_weight, idx2, n_rows, d)
    return out.reshape(b, h, d)
